# baseline (speedup 1.0000x reference)
import functools

import jax
import jax.numpy as jnp
from jax import lax
from jax.experimental import pallas as pl
from jax.experimental.pallas import tpu as pltpu
from jax.experimental.pallas import tpu_sc as plsc

N = 10000
E = 320000
D = 128
NUM_MSG = 6

E_SPLIT = (192000, 128000)

NC, NS = 2, 16
NW = NC * NS
C = 80
CS = 40
NBUF = 5
N_PAD = 10240
ROWS_PER_TILE = N_PAD // NS

BE = 2000


def _pad8(n):
    return (n + 7) // 8 * 8


def _gather_body_for(eh):
    pw = eh // NW
    nch = pw // C
    steps = nch // NBUF

    def body(atom_hbm, src_hbm, dst_hbm, sa_out, ta_out,
             src_v, dst_v, sa_b, ta_b, *sems):
        sem_g = sems[:NBUF]
        sem_w = sems[NBUF:]
        wid = lax.axis_index("s") * NC + lax.axis_index("c")
        base = wid * pw
        pltpu.sync_copy(src_hbm.at[pl.ds(base, pw)], src_v)
        pltpu.sync_copy(dst_hbm.at[pl.ds(base, pw)], dst_v)

        def g_start(j, b):
            off = j * C
            pltpu.async_copy(atom_hbm.at[src_v.at[pl.ds(off, C)]], sa_b.at[b],
                             sem_g[b])
            pltpu.async_copy(atom_hbm.at[dst_v.at[pl.ds(off, C)]], ta_b.at[b],
                             sem_g[b])

        def g_wait(j, b):
            off = j * C
            pltpu.make_async_copy(atom_hbm.at[src_v.at[pl.ds(off, C)]],
                                  sa_b.at[b], sem_g[b]).wait()
            pltpu.make_async_copy(atom_hbm.at[dst_v.at[pl.ds(off, C)]],
                                  ta_b.at[b], sem_g[b]).wait()

        def w_start(j, b):
            off = j * C
            pltpu.async_copy(sa_b.at[b], sa_out.at[pl.ds(base + off, C)],
                             sem_w[b])
            pltpu.async_copy(ta_b.at[b], ta_out.at[pl.ds(base + off, C)],
                             sem_w[b])

        def w_wait(j, b):
            off = j * C
            pltpu.make_async_copy(sa_b.at[b], sa_out.at[pl.ds(base + off, C)],
                                  sem_w[b]).wait()
            pltpu.make_async_copy(ta_b.at[b], ta_out.at[pl.ds(base + off, C)],
                                  sem_w[b]).wait()

        for b in range(NBUF):
            g_start(b, b)

        def loop(t, carry):
            for b in range(NBUF):
                j = t * NBUF + b
                g_wait(j, b)
                w_start(j, b)
            for b in range(NBUF):
                j = t * NBUF + b
                w_wait(j, b)
                g_start(j + NBUF, b)
            return carry

        lax.fori_loop(0, steps - 1, loop, 0)
        last = (steps - 1) * NBUF
        for b in range(NBUF):
            g_wait(last + b, b)
            w_start(last + b, b)
        for b in range(NBUF):
            w_wait(last + b, b)

    return body


@functools.lru_cache(maxsize=None)
def _make_gather_call(eh):
    pw = eh // NW
    return functools.partial(
        pl.kernel,
        out_type=[jax.ShapeDtypeStruct((eh, D), jnp.float32),
                  jax.ShapeDtypeStruct((eh, D), jnp.float32)],
        mesh=plsc.VectorSubcoreMesh(core_axis_name="c", subcore_axis_name="s"),
        scratch_types=[
            pltpu.VMEM((pw,), jnp.int32),
            pltpu.VMEM((pw,), jnp.int32),
            pltpu.VMEM((NBUF, C, D), jnp.float32),
            pltpu.VMEM((NBUF, C, D), jnp.float32),
        ] + [pltpu.SemaphoreType.DMA] * (2 * NBUF),
    )(_gather_body_for(eh))


def _sc_gather(atom, src, dst):
    return _make_gather_call(src.shape[0])(atom, src, dst)


def _scatter_body_for(eh):
    pw = eh // NW
    nchs = pw // CS
    steps_s = nchs // NBUF

    def body(m_hbm, dst3_hbm, zeros_hbm, out_hbm, dst_v, m_buf, shared, *sems):
        sem_l = sems[:NBUF]
        sem_s = sems[NBUF:]
        c = lax.axis_index("c")
        s = lax.axis_index("s")
        wid = s * NC + c
        r0 = s * ROWS_PER_TILE
        pltpu.sync_copy(zeros_hbm.at[pl.ds(r0, ROWS_PER_TILE)],
                        shared.at[pl.ds(r0, ROWS_PER_TILE)])
        plsc.subcore_barrier()

        def l_start(j, b):
            pltpu.async_copy(m_hbm.at[pl.ds(wid * pw + j * CS, CS)],
                             m_buf.at[b], sem_l[b])
            pltpu.async_copy(dst3_hbm.at[wid].at[j], dst_v.at[b], sem_l[b])

        def l_wait(j, b):
            pltpu.make_async_copy(m_hbm.at[pl.ds(wid * pw + j * CS, CS)],
                                  m_buf.at[b], sem_l[b]).wait()
            pltpu.make_async_copy(dst3_hbm.at[wid].at[j], dst_v.at[b],
                                  sem_l[b]).wait()

        def s_start(j, b):
            pltpu.async_copy(m_buf.at[b], shared.at[dst_v.at[b]], sem_s[b],
                             add=True)

        def s_wait(j, b):
            pltpu.make_async_copy(m_buf.at[b], shared.at[dst_v.at[b]],
                                  sem_s[b]).wait()

        for b in range(NBUF):
            l_start(b, b)

        def loop(t, carry):
            for b in range(NBUF):
                j = t * NBUF + b
                l_wait(j, b)
                s_start(j, b)
            for b in range(NBUF):
                j = t * NBUF + b
                s_wait(j, b)
                l_start(j + NBUF, b)
            return carry

        lax.fori_loop(0, steps_s - 1, loop, 0)
        last = (steps_s - 1) * NBUF
        for b in range(NBUF):
            l_wait(last + b, b)
            s_start(last + b, b)
        for b in range(NBUF):
            s_wait(last + b, b)
        plsc.subcore_barrier()
        pltpu.sync_copy(shared.at[pl.ds(r0, ROWS_PER_TILE)],
                        out_hbm.at[c].at[pl.ds(r0, ROWS_PER_TILE)])

    return body


@functools.lru_cache(maxsize=None)
def _make_scatter_call(eh):
    pw = eh // NW
    nchs = pw // CS
    return functools.partial(
        pl.kernel,
        out_type=jax.ShapeDtypeStruct((NC, N_PAD, D), jnp.float32),
        mesh=plsc.VectorSubcoreMesh(core_axis_name="c", subcore_axis_name="s"),
        scratch_types=[
            pltpu.VMEM((NBUF, CS), jnp.int32),
            pltpu.VMEM((NBUF, CS, D), jnp.float32),
            pltpu.VMEM_SHARED((N_PAD, D), jnp.float32),
        ] + [pltpu.SemaphoreType.DMA] * (2 * NBUF),
    )(_scatter_body_for(eh))


def _sc_scatter(m, dst3, zeros_nd):
    return _make_scatter_call(m.shape[0])(m, dst3, zeros_nd)


def _embed_body(site_ref, emb_ref, out_ref):
    site_col = site_ref[...]
    iota = lax.broadcasted_iota(jnp.int32, (N, D), 1)
    oh = (iota == site_col).astype(jnp.float32)
    out_ref[...] = jnp.dot(oh, emb_ref[...], preferred_element_type=jnp.float32, precision=lax.Precision.HIGHEST)


def _embed(site_col, emb_pad):
    return pl.pallas_call(
        _embed_body,
        out_shape=jax.ShapeDtypeStruct((N, D), jnp.float32),
    )(site_col, emb_pad)


def _edge_body_common(bond, d, sa, ta, wb, ws, wt, eb1, ew2, eb2,
                      us, ub, nb1, nw2, nb2, bond_out_ref, m_out_ref):
    f32 = jnp.float32
    h = jnp.dot(bond, wb, preferred_element_type=f32, precision=lax.Precision.HIGHEST)
    h += jnp.dot(sa, ws, preferred_element_type=f32, precision=lax.Precision.HIGHEST)
    h += jnp.dot(ta, wt, preferred_element_type=f32, precision=lax.Precision.HIGHEST)
    h = jax.nn.relu(h + eb1)
    bondn = bond + jnp.dot(h, ew2, preferred_element_type=f32, precision=lax.Precision.HIGHEST) + eb2
    bond_out_ref[...] = bondn
    m1 = jnp.dot(sa, us, preferred_element_type=f32, precision=lax.Precision.HIGHEST)
    m1 += jnp.dot(bondn, ub, preferred_element_type=f32, precision=lax.Precision.HIGHEST)
    m1 = jax.nn.relu(m1 + nb1)
    m = jnp.dot(m1, nw2, preferred_element_type=f32, precision=lax.Precision.HIGHEST) + nb2
    maskf = (d != 0.0).astype(f32)
    m_out_ref[...] = m * maskf


def _edge_body(d_ref, bond_ref, sa_ref, ta_ref, wb, ws, wt, eb1, ew2, eb2,
               us, ub, nb1, nw2, nb2, bond_out_ref, m_out_ref):
    _edge_body_common(bond_ref[...], d_ref[...], sa_ref[...], ta_ref[...],
                      wb[...], ws[...], wt[...], eb1[...], ew2[...], eb2[...],
                      us[...], ub[...], nb1[...], nw2[...], nb2[...],
                      bond_out_ref, m_out_ref)


def _edge_body_first(d_ref, sa_ref, ta_ref, cen_ref, wbond_ref, bbond_ref,
                     gap_ref, wb, ws, wt, eb1, ew2, eb2,
                     us, ub, nb1, nw2, nb2, bond_out_ref, m_out_ref):
    d = d_ref[...]
    d0 = jnp.where(jnp.isnan(d), jnp.zeros_like(d), d)
    rbf = jnp.exp(-gap_ref[0, 0] * (d0 - cen_ref[...]) ** 2)
    bond = (jnp.dot(rbf, wbond_ref[...], preferred_element_type=jnp.float32, precision=lax.Precision.HIGHEST)
            + bbond_ref[...])
    _edge_body_common(bond, d, sa_ref[...], ta_ref[...],
                      wb[...], ws[...], wt[...], eb1[...], ew2[...], eb2[...],
                      us[...], ub[...], nb1[...], nw2[...], nb2[...],
                      bond_out_ref, m_out_ref)


def _full(shape):
    return pl.BlockSpec(shape, lambda i: (0,) * len(shape))


def _edge_out(eh):
    return [jax.ShapeDtypeStruct((eh, D), jnp.float32),
            jax.ShapeDtypeStruct((eh, D), jnp.float32)]


def _edge_out_specs():
    return [pl.BlockSpec((BE, D), lambda i: (i, 0)),
            pl.BlockSpec((BE, D), lambda i: (i, 0))]


def _w_specs():
    return [_full((D, 2 * D)), _full((D, 2 * D)), _full((D, 2 * D)),
            _full((1, 2 * D)), _full((2 * D, D)), _full((1, D)),
            _full((D, 2 * D)), _full((D, 2 * D)), _full((1, 2 * D)),
            _full((2 * D, D)), _full((1, D))]


def _edge_layer(dist2, bond, sa, ta, wts, blk0, eh):
    off = lambda i: (i + blk0, 0)
    return pl.pallas_call(
        _edge_body,
        grid=(eh // BE,),
        in_specs=[pl.BlockSpec((BE, 1), off),
                  pl.BlockSpec((BE, D), lambda i: (i, 0)),
                  pl.BlockSpec((BE, D), lambda i: (i, 0)),
                  pl.BlockSpec((BE, D), lambda i: (i, 0))] + _w_specs(),
        out_specs=_edge_out_specs(),
        out_shape=_edge_out(eh),
    )(dist2, bond, sa, ta, *wts)


def _edge_layer_first(dist2, sa, ta, cen, wbond, bbond, gap2, wts, blk0, eh):
    off = lambda i: (i + blk0, 0)
    return pl.pallas_call(
        _edge_body_first,
        grid=(eh // BE,),
        in_specs=[pl.BlockSpec((BE, 1), off),
                  pl.BlockSpec((BE, D), lambda i: (i, 0)),
                  pl.BlockSpec((BE, D), lambda i: (i, 0)),
                  _full((1, D)), _full((D, D)), _full((1, D)),
                  pl.BlockSpec(memory_space=pltpu.SMEM)] + _w_specs(),
        out_specs=_edge_out_specs(),
        out_shape=_edge_out(eh),
    )(dist2, sa, ta, cen, wbond, bbond, gap2, *wts)


def _post_body(atom_ref, p_ref, q_ref, pw1, pb1, pw2, pb2, out_ref):
    f32 = jnp.float32
    agg = (p_ref[0, :N] + p_ref[1, :N]) + (q_ref[0, :N] + q_ref[1, :N])
    a = jax.nn.relu(jnp.dot(agg, pw1[...], preferred_element_type=f32, precision=lax.Precision.HIGHEST) + pb1[...])
    a = jnp.dot(a, pw2[...], preferred_element_type=f32, precision=lax.Precision.HIGHEST) + pb2[...]
    out_ref[...] = atom_ref[...] + a


def _post(atom, p0, p1, pwts):
    return pl.pallas_call(
        _post_body,
        out_shape=jax.ShapeDtypeStruct((N, D), jnp.float32),
    )(atom, p0, p1, *pwts)


def _post_final_body(atom_ref, p_ref, q_ref, pw1, pb1, pw2, pb2,
                     site_ref, woff_t_ref, boff_ref, embmean_t_ref, out_ref):
    f32 = jnp.float32
    agg = (p_ref[0, :N] + p_ref[1, :N]) + (q_ref[0, :N] + q_ref[1, :N])
    a = jax.nn.relu(jnp.dot(agg, pw1[...], preferred_element_type=f32, precision=lax.Precision.HIGHEST) + pb1[...])
    a = jnp.dot(a, pw2[...], preferred_element_type=f32, precision=lax.Precision.HIGHEST) + pb2[...]
    atomn = atom_ref[...] + a
    site_col = site_ref[...]
    iota = lax.broadcasted_iota(jnp.int32, (N, D), 1)
    oh = (iota == site_col).astype(f32)
    mn = jnp.sum(oh * embmean_t_ref[...], axis=1, keepdims=True)
    val = mn + jnp.sum(atomn * woff_t_ref[...], axis=1, keepdims=True) + boff_ref[0, 0]
    maskf = (site_col != 0).astype(f32)
    num = jnp.sum(val * maskf)
    den = jnp.maximum(jnp.sum(maskf), 1.0)
    out_ref[...] = jnp.full((1, 1), num / den, dtype=f32)


def _post_final(atom, p0, p1, pwts, site_col, woff_t, boff2, embmean_t):
    return pl.pallas_call(
        _post_final_body,
        out_shape=jax.ShapeDtypeStruct((1, 1), jnp.float32),
    )(atom, p0, p1, *pwts, site_col, woff_t, boff2, embmean_t)


def kernel(site, distance, connectivity, emb_atom, emb_mean, centers, gap,
           w_bond, b_bond, w_off, b_off,
           ew1, eb1, ew2, eb2, nw1, nb1, nw2, nb2, pw1, pb1, pw2, pb2):
    f32 = jnp.float32
    site_col = site.reshape(N, 1)
    dist2 = distance.reshape(E, 1)
    conn = connectivity.reshape(E, 2)
    dst = conn[:, 0]
    src = conn[:, 1]
    e0 = E_SPLIT[0]
    src_h = (src[:e0], src[e0:])
    dst_h = (dst[:e0], dst[e0:])

    def _dst3(x):
        nchs = x.shape[0] // NW // CS
        x3 = x.reshape(NW, nchs, CS)
        return jnp.pad(x3, ((0, 0), (0, _pad8(nchs) - nchs), (0, 0)))

    dst3_h = (_dst3(dst[:e0]), _dst3(dst[e0:]))
    emb_pad = jnp.zeros((D, D), f32).at[: emb_atom.shape[0]].set(emb_atom)
    embmean_t = jnp.zeros((1, D), f32).at[0, : emb_mean.shape[0]].set(emb_mean[:, 0])
    zeros_nd = jnp.zeros((N_PAD, D), f32)
    cen = centers.reshape(1, D)
    gap2 = gap.reshape(1, 1)
    bbond = b_bond.reshape(1, D)
    woff_t = w_off.reshape(1, D)
    boff2 = b_off.reshape(1, 1)

    atom = _embed(site_col, emb_pad)
    bond = [None, None]
    out = None
    for l in range(NUM_MSG):
        wts = (ew1[l, :D], ew1[l, D:2 * D], ew1[l, 2 * D:],
               eb1[l].reshape(1, 2 * D), ew2[l], eb2[l].reshape(1, D),
               nw1[l, :D], nw1[l, D:],
               nb1[l].reshape(1, 2 * D), nw2[l], nb2[l].reshape(1, D))
        pwts = (pw1[l], pb1[l].reshape(1, 2 * D), pw2[l], pb2[l].reshape(1, D))
        m = [None, None]
        parts = [None, None]
        sata = [None, None]
        for h in (0, 1):
            sata[h] = _sc_gather(atom, src_h[h], dst_h[h])
        for h in (0, 1):
            sa, ta = sata[h]
            blk0 = (0, e0 // BE)[h]
            eh = E_SPLIT[h]
            if l == 0:
                bond[h], m[h] = _edge_layer_first(dist2, sa, ta, cen, w_bond,
                                                  bbond, gap2, wts, blk0, eh)
            else:
                bond[h], m[h] = _edge_layer(dist2, bond[h], sa, ta, wts,
                                            blk0, eh)
            parts[h] = _sc_scatter(m[h], dst3_h[h], zeros_nd)
        if l < NUM_MSG - 1:
            atom = _post(atom, parts[0], parts[1], pwts)
        else:
            out = _post_final(atom, parts[0], parts[1], pwts, site_col,
                              woff_t, boff2, embmean_t)
    return out


# speedup vs baseline: 4.8200x; 4.8200x over previous
import functools

import jax
import jax.numpy as jnp
from jax import lax
from jax.experimental import pallas as pl
from jax.experimental.pallas import tpu as pltpu
from jax.experimental.pallas import tpu_sc as plsc

N = 10000
E = 320000
D = 128
NUM_MSG = 6

E_SPLIT = (192000, 128000)

NC, NS = 2, 16
NW = NC * NS
C = 80
CS = 40
NBUF = 5
N_PAD = 10240
ROWS_PER_TILE = N_PAD // NS

BE = 2000


def _pad8(n):
    return (n + 7) // 8 * 8


def _gather_body_for(eh):
    pw = eh // NW
    nch = pw // C
    steps = nch // NBUF

    def body(atom_hbm, src_hbm, dst_hbm, sa_out, ta_out,
             src_v, dst_v, sa_b, ta_b, *sems):
        sem_g = sems[:NBUF]
        sem_w = sems[NBUF:]
        wid = lax.axis_index("s") * NC + lax.axis_index("c")
        base = wid * pw
        pltpu.sync_copy(src_hbm.at[pl.ds(base, pw)], src_v)
        pltpu.sync_copy(dst_hbm.at[pl.ds(base, pw)], dst_v)

        def g_start(j, b):
            off = j * C
            pltpu.async_copy(atom_hbm.at[src_v.at[pl.ds(off, C)]], sa_b.at[b],
                             sem_g[b])
            pltpu.async_copy(atom_hbm.at[dst_v.at[pl.ds(off, C)]], ta_b.at[b],
                             sem_g[b])

        def g_wait(j, b):
            off = j * C
            pltpu.make_async_copy(atom_hbm.at[src_v.at[pl.ds(off, C)]],
                                  sa_b.at[b], sem_g[b]).wait()
            pltpu.make_async_copy(atom_hbm.at[dst_v.at[pl.ds(off, C)]],
                                  ta_b.at[b], sem_g[b]).wait()

        def w_start(j, b):
            off = j * C
            pltpu.async_copy(sa_b.at[b], sa_out.at[pl.ds(base + off, C)],
                             sem_w[b])
            pltpu.async_copy(ta_b.at[b], ta_out.at[pl.ds(base + off, C)],
                             sem_w[b])

        def w_wait(j, b):
            off = j * C
            pltpu.make_async_copy(sa_b.at[b], sa_out.at[pl.ds(base + off, C)],
                                  sem_w[b]).wait()
            pltpu.make_async_copy(ta_b.at[b], ta_out.at[pl.ds(base + off, C)],
                                  sem_w[b]).wait()

        for b in range(NBUF):
            g_start(b, b)

        def loop(t, carry):
            for b in range(NBUF):
                j = t * NBUF + b
                g_wait(j, b)
                w_start(j, b)
            for b in range(NBUF):
                j = t * NBUF + b
                w_wait(j, b)
                g_start(j + NBUF, b)
            return carry

        lax.fori_loop(0, steps - 1, loop, 0)
        last = (steps - 1) * NBUF
        for b in range(NBUF):
            g_wait(last + b, b)
            w_start(last + b, b)
        for b in range(NBUF):
            w_wait(last + b, b)

    return body


@functools.lru_cache(maxsize=None)
def _make_gather_call(eh):
    pw = eh // NW
    return functools.partial(
        pl.kernel,
        out_type=[jax.ShapeDtypeStruct((eh, D), jnp.float32),
                  jax.ShapeDtypeStruct((eh, D), jnp.float32)],
        mesh=plsc.VectorSubcoreMesh(core_axis_name="c", subcore_axis_name="s"),
        scratch_types=[
            pltpu.VMEM((pw,), jnp.int32),
            pltpu.VMEM((pw,), jnp.int32),
            pltpu.VMEM((NBUF, C, D), jnp.float32),
            pltpu.VMEM((NBUF, C, D), jnp.float32),
        ] + [pltpu.SemaphoreType.DMA] * (2 * NBUF),
    )(_gather_body_for(eh))


def _sc_gather(atom, src, dst):
    return _make_gather_call(src.shape[0])(atom, src, dst)


def _scatter_body_for(eh):
    pw = eh // NW
    nchs = pw // CS
    steps_s = nchs // NBUF

    def body(m_hbm, dst3_hbm, zeros_hbm, out_hbm, dst_v, m_buf, shared, *sems):
        sem_l = sems[:NBUF]
        sem_s = sems[NBUF:]
        c = lax.axis_index("c")
        s = lax.axis_index("s")
        wid = s * NC + c
        r0 = s * ROWS_PER_TILE
        pltpu.sync_copy(zeros_hbm.at[pl.ds(r0, ROWS_PER_TILE)],
                        shared.at[pl.ds(r0, ROWS_PER_TILE)])
        plsc.subcore_barrier()

        def l_start(j, b):
            pltpu.async_copy(m_hbm.at[pl.ds(wid * pw + j * CS, CS)],
                             m_buf.at[b], sem_l[b])
            pltpu.async_copy(dst3_hbm.at[wid].at[j], dst_v.at[b], sem_l[b])

        def l_wait(j, b):
            pltpu.make_async_copy(m_hbm.at[pl.ds(wid * pw + j * CS, CS)],
                                  m_buf.at[b], sem_l[b]).wait()
            pltpu.make_async_copy(dst3_hbm.at[wid].at[j], dst_v.at[b],
                                  sem_l[b]).wait()

        def s_start(j, b):
            pltpu.async_copy(m_buf.at[b], shared.at[dst_v.at[b]], sem_s[b],
                             add=True)

        def s_wait(j, b):
            pltpu.make_async_copy(m_buf.at[b], shared.at[dst_v.at[b]],
                                  sem_s[b]).wait()

        for b in range(NBUF):
            l_start(b, b)

        def loop(t, carry):
            for b in range(NBUF):
                j = t * NBUF + b
                l_wait(j, b)
                s_start(j, b)
            for b in range(NBUF):
                j = t * NBUF + b
                s_wait(j, b)
                l_start(j + NBUF, b)
            return carry

        lax.fori_loop(0, steps_s - 1, loop, 0)
        last = (steps_s - 1) * NBUF
        for b in range(NBUF):
            l_wait(last + b, b)
            s_start(last + b, b)
        for b in range(NBUF):
            s_wait(last + b, b)
        plsc.subcore_barrier()
        pltpu.sync_copy(shared.at[pl.ds(r0, ROWS_PER_TILE)],
                        out_hbm.at[c].at[pl.ds(r0, ROWS_PER_TILE)])

    return body


@functools.lru_cache(maxsize=None)
def _make_scatter_call(eh):
    pw = eh // NW
    nchs = pw // CS
    return functools.partial(
        pl.kernel,
        out_type=jax.ShapeDtypeStruct((NC, N_PAD, D), jnp.float32),
        mesh=plsc.VectorSubcoreMesh(core_axis_name="c", subcore_axis_name="s"),
        scratch_types=[
            pltpu.VMEM((NBUF, CS), jnp.int32),
            pltpu.VMEM((NBUF, CS, D), jnp.float32),
            pltpu.VMEM_SHARED((N_PAD, D), jnp.float32),
        ] + [pltpu.SemaphoreType.DMA] * (2 * NBUF),
    )(_scatter_body_for(eh))


def _sc_scatter(m, dst3, zeros_nd):
    return _make_scatter_call(m.shape[0])(m, dst3, zeros_nd)


def _embed_body(site_ref, emb_ref, out_ref):
    site_col = site_ref[...]
    iota = lax.broadcasted_iota(jnp.int32, (N, D), 1)
    oh = (iota == site_col).astype(jnp.float32)
    out_ref[...] = jnp.dot(oh, emb_ref[...], preferred_element_type=jnp.float32,
                           precision=lax.Precision.HIGHEST)


def _embed(site_col, emb_pad):
    return pl.pallas_call(
        _embed_body,
        out_shape=jax.ShapeDtypeStruct((N, D), jnp.float32),
    )(site_col, emb_pad)


def _edge_body_common(bond, d, sa, ta, ew1, eb1, ew2, eb2,
                      nw1, nb1, nw2, nb2, bond_out_ref, m_out_ref):
    f32 = jnp.float32
    h = jnp.concatenate([bond, sa, ta], axis=1)
    h = jax.nn.relu(jnp.dot(h, ew1, preferred_element_type=f32) + eb1)
    bondn = bond + jnp.dot(h, ew2, preferred_element_type=f32) + eb2
    bond_out_ref[...] = bondn
    m1 = jnp.concatenate([sa, bondn], axis=1)
    m1 = jax.nn.relu(jnp.dot(m1, nw1, preferred_element_type=f32) + nb1)
    m = jnp.dot(m1, nw2, preferred_element_type=f32) + nb2
    maskf = (d != 0.0).astype(f32)
    m_out_ref[...] = m * maskf


def _edge_body(d_ref, bond_ref, sa_ref, ta_ref, ew1, eb1, ew2, eb2,
               nw1, nb1, nw2, nb2, bond_out_ref, m_out_ref):
    _edge_body_common(bond_ref[...], d_ref[...], sa_ref[...], ta_ref[...],
                      ew1[...], eb1[...], ew2[...], eb2[...],
                      nw1[...], nb1[...], nw2[...], nb2[...],
                      bond_out_ref, m_out_ref)


def _edge_body_first(d_ref, sa_ref, ta_ref, cen_ref, wbond_ref, bbond_ref,
                     gap_ref, ew1, eb1, ew2, eb2,
                     nw1, nb1, nw2, nb2, bond_out_ref, m_out_ref):
    d = d_ref[...]
    d0 = jnp.where(jnp.isnan(d), jnp.zeros_like(d), d)
    rbf = jnp.exp(-gap_ref[0, 0] * (d0 - cen_ref[...]) ** 2)
    bond = (jnp.dot(rbf, wbond_ref[...], preferred_element_type=jnp.float32)
            + bbond_ref[...])
    _edge_body_common(bond, d, sa_ref[...], ta_ref[...],
                      ew1[...], eb1[...], ew2[...], eb2[...],
                      nw1[...], nb1[...], nw2[...], nb2[...],
                      bond_out_ref, m_out_ref)


def _full(shape):
    return pl.BlockSpec(shape, lambda i: (0,) * len(shape))


def _edge_out(eh):
    return [jax.ShapeDtypeStruct((eh, D), jnp.float32),
            jax.ShapeDtypeStruct((eh, D), jnp.float32)]


def _edge_out_specs():
    return [pl.BlockSpec((BE, D), lambda i: (i, 0)),
            pl.BlockSpec((BE, D), lambda i: (i, 0))]


def _w_specs():
    return [_full((3 * D, 2 * D)), _full((1, 2 * D)), _full((2 * D, D)),
            _full((1, D)), _full((2 * D, 2 * D)), _full((1, 2 * D)),
            _full((2 * D, D)), _full((1, D))]


def _edge_layer(dist2, bond, sa, ta, wts, blk0, eh):
    off = lambda i: (i + blk0, 0)
    return pl.pallas_call(
        _edge_body,
        grid=(eh // BE,),
        in_specs=[pl.BlockSpec((BE, 1), off),
                  pl.BlockSpec((BE, D), lambda i: (i, 0)),
                  pl.BlockSpec((BE, D), lambda i: (i, 0)),
                  pl.BlockSpec((BE, D), lambda i: (i, 0))] + _w_specs(),
        out_specs=_edge_out_specs(),
        out_shape=_edge_out(eh),
    )(dist2, bond, sa, ta, *wts)


def _edge_layer_first(dist2, sa, ta, cen, wbond, bbond, gap2, wts, blk0, eh):
    off = lambda i: (i + blk0, 0)
    return pl.pallas_call(
        _edge_body_first,
        grid=(eh // BE,),
        in_specs=[pl.BlockSpec((BE, 1), off),
                  pl.BlockSpec((BE, D), lambda i: (i, 0)),
                  pl.BlockSpec((BE, D), lambda i: (i, 0)),
                  _full((1, D)), _full((D, D)), _full((1, D)),
                  pl.BlockSpec(memory_space=pltpu.SMEM)] + _w_specs(),
        out_specs=_edge_out_specs(),
        out_shape=_edge_out(eh),
    )(dist2, sa, ta, cen, wbond, bbond, gap2, *wts)


def _post_body(atom_ref, p_ref, q_ref, pw1, pb1, pw2, pb2, out_ref):
    f32 = jnp.float32
    agg = (p_ref[0, :N] + p_ref[1, :N]) + (q_ref[0, :N] + q_ref[1, :N])
    a = jax.nn.relu(jnp.dot(agg, pw1[...], preferred_element_type=f32) + pb1[...])
    a = jnp.dot(a, pw2[...], preferred_element_type=f32) + pb2[...]
    out_ref[...] = atom_ref[...] + a


def _post(atom, p0, p1, pwts):
    return pl.pallas_call(
        _post_body,
        out_shape=jax.ShapeDtypeStruct((N, D), jnp.float32),
    )(atom, p0, p1, *pwts)


def _post_final_body(atom_ref, p_ref, q_ref, pw1, pb1, pw2, pb2,
                     site_ref, woff_t_ref, boff_ref, embmean_t_ref, out_ref):
    f32 = jnp.float32
    agg = (p_ref[0, :N] + p_ref[1, :N]) + (q_ref[0, :N] + q_ref[1, :N])
    a = jax.nn.relu(jnp.dot(agg, pw1[...], preferred_element_type=f32) + pb1[...])
    a = jnp.dot(a, pw2[...], preferred_element_type=f32) + pb2[...]
    atomn = atom_ref[...] + a
    site_col = site_ref[...]
    iota = lax.broadcasted_iota(jnp.int32, (N, D), 1)
    oh = (iota == site_col).astype(f32)
    mn = jnp.sum(oh * embmean_t_ref[...], axis=1, keepdims=True)
    val = mn + jnp.sum(atomn * woff_t_ref[...], axis=1, keepdims=True) + boff_ref[0, 0]
    maskf = (site_col != 0).astype(f32)
    num = jnp.sum(val * maskf)
    den = jnp.maximum(jnp.sum(maskf), 1.0)
    out_ref[...] = jnp.full((1, 1), num / den, dtype=f32)


def _post_final(atom, p0, p1, pwts, site_col, woff_t, boff2, embmean_t):
    return pl.pallas_call(
        _post_final_body,
        out_shape=jax.ShapeDtypeStruct((1, 1), jnp.float32),
    )(atom, p0, p1, *pwts, site_col, woff_t, boff2, embmean_t)


def kernel(site, distance, connectivity, emb_atom, emb_mean, centers, gap,
           w_bond, b_bond, w_off, b_off,
           ew1, eb1, ew2, eb2, nw1, nb1, nw2, nb2, pw1, pb1, pw2, pb2):
    f32 = jnp.float32
    site_col = site.reshape(N, 1)
    dist2 = distance.reshape(E, 1)
    conn = connectivity.reshape(E, 2)
    dst = conn[:, 0]
    src = conn[:, 1]
    e0 = E_SPLIT[0]
    src_h = (src[:e0], src[e0:])
    dst_h = (dst[:e0], dst[e0:])

    def _dst3(x):
        nchs = x.shape[0] // NW // CS
        x3 = x.reshape(NW, nchs, CS)
        return jnp.pad(x3, ((0, 0), (0, _pad8(nchs) - nchs), (0, 0)))

    dst3_h = (_dst3(dst[:e0]), _dst3(dst[e0:]))
    emb_pad = jnp.zeros((D, D), f32).at[: emb_atom.shape[0]].set(emb_atom)
    embmean_t = jnp.zeros((1, D), f32).at[0, : emb_mean.shape[0]].set(emb_mean[:, 0])
    zeros_nd = jnp.zeros((N_PAD, D), f32)
    cen = centers.reshape(1, D)
    gap2 = gap.reshape(1, 1)
    bbond = b_bond.reshape(1, D)
    woff_t = w_off.reshape(1, D)
    boff2 = b_off.reshape(1, 1)

    atom = _embed(site_col, emb_pad)
    bond = [None, None]
    out = None
    for l in range(NUM_MSG):
        wts = (ew1[l], eb1[l].reshape(1, 2 * D), ew2[l],
               eb2[l].reshape(1, D), nw1[l],
               nb1[l].reshape(1, 2 * D), nw2[l], nb2[l].reshape(1, D))
        pwts = (pw1[l], pb1[l].reshape(1, 2 * D), pw2[l], pb2[l].reshape(1, D))
        m = [None, None]
        parts = [None, None]
        sata = [None, None]
        for h in (0, 1):
            sata[h] = _sc_gather(atom, src_h[h], dst_h[h])
        for h in (0, 1):
            sa, ta = sata[h]
            blk0 = (0, e0 // BE)[h]
            eh = E_SPLIT[h]
            if l == 0:
                bond[h], m[h] = _edge_layer_first(dist2, sa, ta, cen, w_bond,
                                                  bbond, gap2, wts, blk0, eh)
            else:
                bond[h], m[h] = _edge_layer(dist2, bond[h], sa, ta, wts,
                                            blk0, eh)
            parts[h] = _sc_scatter(m[h], dst3_h[h], zeros_nd)
        if l < NUM_MSG - 1:
            atom = _post(atom, parts[0], parts[1], pwts)
        else:
            out = _post_final(atom, parts[0], parts[1], pwts, site_col,
                              woff_t, boff2, embmean_t)
    return out
